# Initial kernel scaffold; baseline (speedup 1.0000x reference)
import functools
import math

import jax
import jax.numpy as jnp
from jax import lax
from jax.experimental import pallas as pl
from jax.experimental.pallas import tpu as pltpu
from jax.experimental.pallas import tpu_sc as plsc

_BN_SCALE = 1.0 / math.sqrt(1.0 + 1e-5)

_N = 50000
_E = 800000
_BLK = 1000
_GRID = _N // _BLK
_NT = 16
_NS = 50016
_RPT = _NS // _NT
_EPAD = 819200
_TRASH = _N


def _full(spec_shape):
    nd = len(spec_shape)
    return pl.BlockSpec(spec_shape, lambda i, _n=nd: (0,) * _n)


def _rows(w):
    return pl.BlockSpec((_BLK, w), lambda i: (i, 0))


def _pre_body(feat, ctx, w1t, b1, w2t, b2, awt, ab, imp, lng, lnb,
              e1t, eb1, e2t, eb2, wlt, wrt, sb, p_ref, r_ref):
    ce = jax.nn.relu(jnp.dot(ctx[...], w1t[...], preferred_element_type=jnp.float32) + b1[...])
    ce = jnp.dot(ce, w2t[...], preferred_element_type=jnp.float32) + b2[...]
    logits = jnp.dot(ce, awt[...], preferred_element_type=jnp.float32) + ab[...]
    att = jax.nn.softmax(logits, axis=-1)
    x = feat[...] * (att * imp[...])
    m = jnp.mean(x, axis=-1, keepdims=True)
    v = jnp.mean((x - m) * (x - m), axis=-1, keepdims=True)
    x = (x - m) / jnp.sqrt(v + 1e-5) * lng[...] + lnb[...]
    x = jax.nn.relu(jnp.dot(x, e1t[...], preferred_element_type=jnp.float32) + eb1[...])
    x = jax.nn.relu(jnp.dot(x, e2t[...], preferred_element_type=jnp.float32) + eb2[...])
    p_ref[...] = jnp.dot(x, wlt[...], preferred_element_type=jnp.float32)
    r_ref[...] = jnp.dot(x, wrt[...], preferred_element_type=jnp.float32) + sb[...]


def _mid1_body(sa, sb_, r, c0, c1, bng, bnb, wlt, wrt, b_, p_ref, r_ref, inv_ref):
    cnt = c0[...][:, 0:1] + c1[...][:, 0:1]
    inv = 1.0 / jnp.maximum(cnt, 1.0)
    s = jnp.concatenate([sa[...], sb_[...]], axis=1)
    x = jax.nn.relu((s * inv + r[...]) * (bng[...] * _BN_SCALE) + bnb[...])
    p_ref[...] = jnp.dot(x, wlt[...], preferred_element_type=jnp.float32)
    r_ref[...] = jnp.dot(x, wrt[...], preferred_element_type=jnp.float32) + b_[...]
    inv_ref[...] = inv


def _mid2_body(sa, sb_, r, inv, bng, bnb, wlt, wrt, b_, p_ref, r_ref):
    s = jnp.concatenate([sa[...], sb_[...]], axis=1)
    x = jax.nn.relu((s * inv[...] + r[...]) * (bng[...] * _BN_SCALE) + bnb[...])
    p_ref[...] = jnp.dot(x, wlt[...], preferred_element_type=jnp.float32)
    r_ref[...] = jnp.dot(x, wrt[...], preferred_element_type=jnp.float32) + b_[...]


def _final_body(sa, sb_, r, inv, bng, bnb, w1t, b1, w2t, b2, out_ref):
    s = jnp.concatenate([sa[...], sb_[...]], axis=1)
    x = jax.nn.relu((s * inv[...] + r[...]) * (bng[...] * _BN_SCALE) + bnb[...])
    h = jax.nn.relu(jnp.dot(x, w1t[...], preferred_element_type=jnp.float32) + b1[...])
    z = jnp.dot(h, w2t[...], preferred_element_type=jnp.float32) + b2[...]
    out_ref[...] = jax.nn.sigmoid(z)


def _make_agg(wh, ki):
    mesh = plsc.VectorSubcoreMesh(core_axis_name="c", subcore_axis_name="s")
    ept = _EPAD // _NT
    n_outer = ept // (ki * 128)

    @functools.partial(
        pl.kernel,
        out_type=jax.ShapeDtypeStruct((2, _NS, wh), jnp.float32),
        mesh=mesh,
        scratch_types=[
            pltpu.VMEM((ki, 128), jnp.int32),
            pltpu.VMEM((ki, 128), jnp.int32),
            pltpu.VMEM((128, wh), jnp.float32),
            pltpu.VMEM_SHARED((_NS, wh), jnp.float32),
            pltpu.SemaphoreType.DMA,
        ],
    )
    def agg(p_hbm, srcA, srcB, dst, zeros_hbm, out_hbm, src_v, dst_v, rows_v, acc, sem):
        c = lax.axis_index("c")
        s = lax.axis_index("s")
        pltpu.sync_copy(zeros_hbm, acc.at[pl.ds(s * _RPT, _RPT)])
        plsc.subcore_barrier()
        row0 = s * (ept // 128)

        def outer(k, carry):
            b = row0 + k * ki

            @pl.when(c == 0)
            def _():
                pltpu.sync_copy(srcA.at[pl.ds(b, ki)], src_v)

            @pl.when(c == 1)
            def _():
                pltpu.sync_copy(srcB.at[pl.ds(b, ki)], src_v)

            pltpu.sync_copy(dst.at[pl.ds(b, ki)], dst_v)
            for j in range(ki):
                pltpu.async_copy(p_hbm.at[src_v.at[j]], rows_v, sem).wait()
                pltpu.sync_copy(rows_v, acc.at[dst_v.at[j]], add=True)
            return carry

        lax.fori_loop(0, n_outer, outer, 0)
        plsc.subcore_barrier()
        pltpu.sync_copy(acc.at[pl.ds(s * _RPT, _RPT)],
                        out_hbm.at[c, pl.ds(s * _RPT, _RPT)])

    return agg


def _make_cnt(ki):
    mesh = plsc.VectorSubcoreMesh(core_axis_name="c", subcore_axis_name="s")
    ept = _EPAD // 2 // _NT
    n_outer = ept // (ki * 128)

    @functools.partial(
        pl.kernel,
        out_type=jax.ShapeDtypeStruct((2, _NS, 16), jnp.float32),
        mesh=mesh,
        scratch_types=[
            pltpu.VMEM((ki, 128), jnp.int32),
            pltpu.VMEM((128, 16), jnp.float32),
            pltpu.VMEM_SHARED((_NS, 16), jnp.float32),
        ],
    )
    def cnt_k(dst, ones_hbm, zeros_hbm, out_hbm, dst_v, ones_v, acc):
        c = lax.axis_index("c")
        s = lax.axis_index("s")
        pltpu.sync_copy(ones_hbm, ones_v)
        pltpu.sync_copy(zeros_hbm, acc.at[pl.ds(s * _RPT, _RPT)])
        plsc.subcore_barrier()
        row0 = (c * (_EPAD // 2) + s * ept) // 128

        def outer(k, carry):
            b = row0 + k * ki
            pltpu.sync_copy(dst.at[pl.ds(b, ki)], dst_v)
            for j in range(ki):
                pltpu.sync_copy(ones_v, acc.at[dst_v.at[j]], add=True)
            return carry

        lax.fori_loop(0, n_outer, outer, 0)
        plsc.subcore_barrier()
        pltpu.sync_copy(acc.at[pl.ds(s * _RPT, _RPT)],
                        out_hbm.at[c, pl.ds(s * _RPT, _RPT)])

    return cnt_k


def kernel(accessibility_features, edge_index, context_features, ctx_W1, ctx_b1,
           ctx_W2, ctx_b2, att_W, att_b, base_importance, ln_g, ln_b,
           enc_W1, enc_b1, enc_W2, enc_b2, sage1_Wl, sage1_Wr, sage1_b,
           bn1_g, bn1_b, sage2_Wl, sage2_Wr, sage2_b, bn2_g, bn2_b,
           sage3_Wl, sage3_Wr, sage3_b, bn3_g, bn3_b,
           svi_W1, svi_b1, svi_W2, svi_b2):
    f32 = jnp.float32
    r1c = lambda a: a.reshape(1, -1).astype(f32)

    ctx = jnp.pad(context_features, ((0, 0), (0, 3)))
    w1t = jnp.pad(ctx_W1.T, ((0, 3), (0, 0)))
    src = edge_index[0]
    dst = edge_index[1]
    pad = _EPAD - _E
    srcA = jnp.concatenate([src * 2, jnp.zeros((pad,), jnp.int32)]).reshape(_EPAD // 128, 128)
    srcB = jnp.concatenate([src * 2 + 1, jnp.zeros((pad,), jnp.int32)]).reshape(_EPAD // 128, 128)
    dstP = jnp.concatenate([dst, jnp.full((pad,), _TRASH, jnp.int32)]).reshape(_EPAD // 128, 128)
    zeros32 = jnp.zeros((_RPT, 32), f32)
    zeros16 = jnp.zeros((_RPT, 16), f32)
    ones16 = jnp.ones((128, 16), f32)

    p1, r1 = pl.pallas_call(
        _pre_body,
        grid=(_GRID,),
        in_specs=[_rows(128), _rows(8), _full((8, 32)), _full((1, 32)),
                  _full((32, 32)), _full((1, 32)), _full((32, 128)), _full((1, 128)),
                  _full((1, 128)), _full((1, 128)), _full((1, 128)),
                  _full((128, 64)), _full((1, 64)), _full((64, 64)), _full((1, 64)),
                  _full((64, 64)), _full((64, 64)), _full((1, 64))],
        out_specs=[_rows(64), _rows(64)],
        out_shape=[jax.ShapeDtypeStruct((_N, 64), f32),
                   jax.ShapeDtypeStruct((_N, 64), f32)],
    )(accessibility_features, ctx, w1t, r1c(ctx_b1), ctx_W2.T, r1c(ctx_b2),
      att_W.T, r1c(att_b), r1c(base_importance), r1c(ln_g), r1c(ln_b),
      enc_W1.T, r1c(enc_b1), enc_W2.T, r1c(enc_b2),
      sage1_Wl.T, sage1_Wr.T, r1c(sage1_b))

    cnt2 = _make_cnt(8)(dstP, ones16, zeros16)

    agg64 = _make_agg(32, 16)
    s1 = agg64(p1.reshape(2 * _N, 32), srcA, srcB, dstP, zeros32)

    p2, r2, inv = pl.pallas_call(
        _mid1_body,
        grid=(_GRID,),
        in_specs=[_rows(32), _rows(32), _rows(64), _rows(16), _rows(16),
                  _full((1, 64)), _full((1, 64)),
                  _full((64, 64)), _full((64, 64)), _full((1, 64))],
        out_specs=[_rows(64), _rows(64), _rows(1)],
        out_shape=[jax.ShapeDtypeStruct((_N, 64), f32),
                   jax.ShapeDtypeStruct((_N, 64), f32),
                   jax.ShapeDtypeStruct((_N, 1), f32)],
    )(s1[0], s1[1], r1, cnt2[0], cnt2[1], r1c(bn1_g), r1c(bn1_b),
      sage2_Wl.T, sage2_Wr.T, r1c(sage2_b))

    s2 = agg64(p2.reshape(2 * _N, 32), srcA, srcB, dstP, zeros32)

    p3, r3 = pl.pallas_call(
        _mid2_body,
        grid=(_GRID,),
        in_specs=[_rows(32), _rows(32), _rows(64), _rows(1),
                  _full((1, 64)), _full((1, 64)),
                  _full((64, 32)), _full((64, 32)), _full((1, 32))],
        out_specs=[_rows(32), _rows(32)],
        out_shape=[jax.ShapeDtypeStruct((_N, 32), f32),
                   jax.ShapeDtypeStruct((_N, 32), f32)],
    )(s2[0], s2[1], r2, inv, r1c(bn2_g), r1c(bn2_b),
      sage3_Wl.T, sage3_Wr.T, r1c(sage3_b))

    s3 = _make_agg(16, 16)(p3.reshape(2 * _N, 16), srcA, srcB, dstP, zeros16)

    svi = pl.pallas_call(
        _final_body,
        grid=(_GRID,),
        in_specs=[_rows(16), _rows(16), _rows(32), _rows(1),
                  _full((1, 32)), _full((1, 32)),
                  _full((32, 16)), _full((1, 16)), _full((16, 1)), _full((1, 1))],
        out_specs=[_rows(1)],
        out_shape=[jax.ShapeDtypeStruct((_N, 1), f32)],
    )(s3[0], s3[1], r3, inv, r1c(bn3_g), r1c(bn3_b),
      svi_W1.T, r1c(svi_b1), svi_W2.T, r1c(svi_b2))[0]

    return svi[:, 0]


# speedup vs baseline: 4.2400x; 4.2400x over previous
import functools
import math

import jax
import jax.numpy as jnp
from jax import lax
from jax.experimental import pallas as pl
from jax.experimental.pallas import tpu as pltpu
from jax.experimental.pallas import tpu_sc as plsc

_BN_SCALE = 1.0 / math.sqrt(1.0 + 1e-5)

_N = 50000
_E = 800000
_BLK = 1000
_GRID = _N // _BLK
_NT = 16
_NS = 50048
_RPT = _NS // _NT
_EPAD = 819200
_TRASH = _N


def _full(spec_shape):
    nd = len(spec_shape)
    return pl.BlockSpec(spec_shape, lambda i, _n=nd: (0,) * _n)


def _rows(w):
    return pl.BlockSpec((_BLK, w), lambda i: (i, 0))


def _pre_body(feat, ctx, w1t, b1, w2t, b2, awt, ab, imp, lng, lnb,
              e1t, eb1, e2t, eb2, wlt, wrt, sb, p_ref, r_ref):
    ce = jax.nn.relu(jnp.dot(ctx[...], w1t[...], preferred_element_type=jnp.float32) + b1[...])
    ce = jnp.dot(ce, w2t[...], preferred_element_type=jnp.float32) + b2[...]
    logits = jnp.dot(ce, awt[...], preferred_element_type=jnp.float32) + ab[...]
    att = jax.nn.softmax(logits, axis=-1)
    x = feat[...] * (att * imp[...])
    m = jnp.mean(x, axis=-1, keepdims=True)
    v = jnp.mean((x - m) * (x - m), axis=-1, keepdims=True)
    x = (x - m) / jnp.sqrt(v + 1e-5) * lng[...] + lnb[...]
    x = jax.nn.relu(jnp.dot(x, e1t[...], preferred_element_type=jnp.float32) + eb1[...])
    x = jax.nn.relu(jnp.dot(x, e2t[...], preferred_element_type=jnp.float32) + eb2[...])
    p_ref[...] = jnp.dot(x, wlt[...], preferred_element_type=jnp.float32)
    r_ref[...] = jnp.dot(x, wrt[...], preferred_element_type=jnp.float32) + sb[...]


def _mid1_body(sa, sb_, r, c0, c1, bng, bnb, wlt, wrt, b_, p_ref, r_ref, inv_ref):
    cnt = c0[...][:, 0:1] + c1[...][:, 0:1]
    inv = 1.0 / jnp.maximum(cnt, 1.0)
    s = jnp.concatenate([sa[...], sb_[...]], axis=1)
    x = jax.nn.relu((s * inv + r[...]) * (bng[...] * _BN_SCALE) + bnb[...])
    p_ref[...] = jnp.dot(x, wlt[...], preferred_element_type=jnp.float32)
    r_ref[...] = jnp.dot(x, wrt[...], preferred_element_type=jnp.float32) + b_[...]
    inv_ref[...] = inv


def _mid2_body(sa, sb_, r, inv, bng, bnb, wlt, wrt, b_, p_ref, r_ref):
    s = jnp.concatenate([sa[...], sb_[...]], axis=1)
    x = jax.nn.relu((s * inv[...] + r[...]) * (bng[...] * _BN_SCALE) + bnb[...])
    p_ref[...] = jnp.dot(x, wlt[...], preferred_element_type=jnp.float32)
    r_ref[...] = jnp.dot(x, wrt[...], preferred_element_type=jnp.float32) + b_[...]


def _final_body(sa, sb_, r, inv, bng, bnb, w1t, b1, w2t, b2, out_ref):
    s = jnp.concatenate([sa[...], sb_[...]], axis=1)
    x = jax.nn.relu((s * inv[...] + r[...]) * (bng[...] * _BN_SCALE) + bnb[...])
    h = jax.nn.relu(jnp.dot(x, w1t[...], preferred_element_type=jnp.float32) + b1[...])
    z = jnp.dot(h, w2t[...], preferred_element_type=jnp.float32) + b2[...]
    out_ref[...] = jax.nn.sigmoid(z)


def _make_agg(wh, ki):
    mesh = plsc.VectorSubcoreMesh(core_axis_name="c", subcore_axis_name="s")
    ept = _EPAD // _NT
    n_outer = ept // (ki * 128)

    @functools.partial(
        pl.kernel,
        out_type=jax.ShapeDtypeStruct((2, _NS, wh), jnp.float32),
        mesh=mesh,
        scratch_types=[
            pltpu.VMEM((ki, 128), jnp.int32),
            pltpu.VMEM((ki, 128), jnp.int32),
            pltpu.VMEM((128, wh), jnp.float32),
            pltpu.VMEM_SHARED((_NS, wh), jnp.float32),
            pltpu.SemaphoreType.DMA,
        ],
        compiler_params=pltpu.CompilerParams(use_tc_tiling_on_sc=False),
    )
    def agg(p_hbm, srcA, srcB, dst, zeros_hbm, out_hbm, src_v, dst_v, rows_v, acc, sem):
        c = lax.axis_index("c")
        s = lax.axis_index("s")
        pltpu.sync_copy(zeros_hbm, acc.at[pl.ds(s * _RPT, _RPT)])
        plsc.subcore_barrier()
        row0 = s * (ept // 128)

        def outer(k, carry):
            b = row0 + k * ki

            @pl.when(c == 0)
            def _():
                pltpu.sync_copy(srcA.at[pl.ds(b, ki)], src_v)

            @pl.when(c == 1)
            def _():
                pltpu.sync_copy(srcB.at[pl.ds(b, ki)], src_v)

            pltpu.sync_copy(dst.at[pl.ds(b, ki)], dst_v)
            for j in range(ki):
                pltpu.async_copy(p_hbm.at[src_v.at[j]], rows_v, sem).wait()
                pltpu.sync_copy(rows_v, acc.at[dst_v.at[j]], add=True)
            return carry

        lax.fori_loop(0, n_outer, outer, 0)
        plsc.subcore_barrier()
        pltpu.sync_copy(acc.at[pl.ds(s * _RPT, _RPT)],
                        out_hbm.at[c, pl.ds(s * _RPT, _RPT)])

    return agg


def _make_cnt(ki):
    mesh = plsc.VectorSubcoreMesh(core_axis_name="c", subcore_axis_name="s")
    ept = _EPAD // 2 // _NT
    n_outer = ept // (ki * 128)

    @functools.partial(
        pl.kernel,
        out_type=jax.ShapeDtypeStruct((2, _NS, 16), jnp.float32),
        mesh=mesh,
        scratch_types=[
            pltpu.VMEM((ki, 128), jnp.int32),
            pltpu.VMEM((128, 16), jnp.float32),
            pltpu.VMEM_SHARED((_NS, 16), jnp.float32),
        ],
        compiler_params=pltpu.CompilerParams(use_tc_tiling_on_sc=False),
    )
    def cnt_k(dst, ones_hbm, zeros_hbm, out_hbm, dst_v, ones_v, acc):
        c = lax.axis_index("c")
        s = lax.axis_index("s")
        pltpu.sync_copy(ones_hbm, ones_v)
        pltpu.sync_copy(zeros_hbm, acc.at[pl.ds(s * _RPT, _RPT)])
        plsc.subcore_barrier()
        row0 = (c * (_EPAD // 2) + s * ept) // 128

        def outer(k, carry):
            b = row0 + k * ki
            pltpu.sync_copy(dst.at[pl.ds(b, ki)], dst_v)
            for j in range(ki):
                pltpu.sync_copy(ones_v, acc.at[dst_v.at[j]], add=True)
            return carry

        lax.fori_loop(0, n_outer, outer, 0)
        plsc.subcore_barrier()
        pltpu.sync_copy(acc.at[pl.ds(s * _RPT, _RPT)],
                        out_hbm.at[c, pl.ds(s * _RPT, _RPT)])

    return cnt_k


def kernel(accessibility_features, edge_index, context_features, ctx_W1, ctx_b1,
           ctx_W2, ctx_b2, att_W, att_b, base_importance, ln_g, ln_b,
           enc_W1, enc_b1, enc_W2, enc_b2, sage1_Wl, sage1_Wr, sage1_b,
           bn1_g, bn1_b, sage2_Wl, sage2_Wr, sage2_b, bn2_g, bn2_b,
           sage3_Wl, sage3_Wr, sage3_b, bn3_g, bn3_b,
           svi_W1, svi_b1, svi_W2, svi_b2):
    f32 = jnp.float32
    r1c = lambda a: a.reshape(1, -1).astype(f32)

    ctx = jnp.pad(context_features, ((0, 0), (0, 3)))
    w1t = jnp.pad(ctx_W1.T, ((0, 3), (0, 0)))
    src = edge_index[0]
    dst = edge_index[1]
    pad = _EPAD - _E
    srcA = jnp.concatenate([src * 2, jnp.zeros((pad,), jnp.int32)]).reshape(_EPAD // 128, 128)
    srcB = jnp.concatenate([src * 2 + 1, jnp.zeros((pad,), jnp.int32)]).reshape(_EPAD // 128, 128)
    dstP = jnp.concatenate([dst, jnp.full((pad,), _TRASH, jnp.int32)]).reshape(_EPAD // 128, 128)
    zeros32 = jnp.zeros((_RPT, 32), f32)
    zeros16 = jnp.zeros((_RPT, 16), f32)
    ones16 = jnp.ones((128, 16), f32)

    p1, r1 = pl.pallas_call(
        _pre_body,
        grid=(_GRID,),
        in_specs=[_rows(128), _rows(8), _full((8, 32)), _full((1, 32)),
                  _full((32, 32)), _full((1, 32)), _full((32, 128)), _full((1, 128)),
                  _full((1, 128)), _full((1, 128)), _full((1, 128)),
                  _full((128, 64)), _full((1, 64)), _full((64, 64)), _full((1, 64)),
                  _full((64, 64)), _full((64, 64)), _full((1, 64))],
        out_specs=[_rows(64), _rows(64)],
        out_shape=[jax.ShapeDtypeStruct((_N, 64), f32),
                   jax.ShapeDtypeStruct((_N, 64), f32)],
    )(accessibility_features, ctx, w1t, r1c(ctx_b1), ctx_W2.T, r1c(ctx_b2),
      att_W.T, r1c(att_b), r1c(base_importance), r1c(ln_g), r1c(ln_b),
      enc_W1.T, r1c(enc_b1), enc_W2.T, r1c(enc_b2),
      sage1_Wl.T, sage1_Wr.T, r1c(sage1_b))

    cnt2 = _make_cnt(8)(dstP, ones16, zeros16)

    agg64 = _make_agg(32, 16)
    s1 = agg64(p1.reshape(2 * _N, 32), srcA, srcB, dstP, zeros32)

    p2, r2, inv = pl.pallas_call(
        _mid1_body,
        grid=(_GRID,),
        in_specs=[_rows(32), _rows(32), _rows(64), _rows(16), _rows(16),
                  _full((1, 64)), _full((1, 64)),
                  _full((64, 64)), _full((64, 64)), _full((1, 64))],
        out_specs=[_rows(64), _rows(64), _rows(1)],
        out_shape=[jax.ShapeDtypeStruct((_N, 64), f32),
                   jax.ShapeDtypeStruct((_N, 64), f32),
                   jax.ShapeDtypeStruct((_N, 1), f32)],
    )(s1[0], s1[1], r1, cnt2[0], cnt2[1], r1c(bn1_g), r1c(bn1_b),
      sage2_Wl.T, sage2_Wr.T, r1c(sage2_b))

    s2 = agg64(p2.reshape(2 * _N, 32), srcA, srcB, dstP, zeros32)

    p3, r3 = pl.pallas_call(
        _mid2_body,
        grid=(_GRID,),
        in_specs=[_rows(32), _rows(32), _rows(64), _rows(1),
                  _full((1, 64)), _full((1, 64)),
                  _full((64, 32)), _full((64, 32)), _full((1, 32))],
        out_specs=[_rows(32), _rows(32)],
        out_shape=[jax.ShapeDtypeStruct((_N, 32), f32),
                   jax.ShapeDtypeStruct((_N, 32), f32)],
    )(s2[0], s2[1], r2, inv, r1c(bn2_g), r1c(bn2_b),
      sage3_Wl.T, sage3_Wr.T, r1c(sage3_b))

    s3 = _make_agg(16, 16)(p3.reshape(2 * _N, 16), srcA, srcB, dstP, zeros16)

    svi = pl.pallas_call(
        _final_body,
        grid=(_GRID,),
        in_specs=[_rows(16), _rows(16), _rows(32), _rows(1),
                  _full((1, 32)), _full((1, 32)),
                  _full((32, 16)), _full((1, 16)), _full((16, 1)), _full((1, 1))],
        out_specs=[_rows(1)],
        out_shape=[jax.ShapeDtypeStruct((_N, 1), f32)],
    )(s3[0], s3[1], r3, inv, r1c(bn3_g), r1c(bn3_b),
      svi_W1.T, r1c(svi_b1), svi_W2.T, r1c(svi_b2))[0]

    return svi[:, 0]


# speedup vs baseline: 5.0021x; 1.1797x over previous
import functools
import math

import jax
import jax.numpy as jnp
from jax import lax
from jax.experimental import pallas as pl
from jax.experimental.pallas import tpu as pltpu
from jax.experimental.pallas import tpu_sc as plsc

_BN_SCALE = 1.0 / math.sqrt(1.0 + 1e-5)

_N = 50000
_E = 800000
_BLK = 1000
_GRID = _N // _BLK
_NT = 16
_NS = 50048
_RPT = _NS // _NT
_EPAD = 819200
_TRASH = _N


def _full(spec_shape):
    nd = len(spec_shape)
    return pl.BlockSpec(spec_shape, lambda i, _n=nd: (0,) * _n)


def _rows(w):
    return pl.BlockSpec((_BLK, w), lambda i: (i, 0))


def _pre_body(feat, ctx, w1t, b1, w2t, b2, awt, ab, imp, lng, lnb,
              e1t, eb1, e2t, eb2, wlt, wrt, sb, p_ref, r_ref):
    ce = jax.nn.relu(jnp.dot(ctx[...], w1t[...], preferred_element_type=jnp.float32) + b1[...])
    ce = jnp.dot(ce, w2t[...], preferred_element_type=jnp.float32) + b2[...]
    logits = jnp.dot(ce, awt[...], preferred_element_type=jnp.float32) + ab[...]
    att = jax.nn.softmax(logits, axis=-1)
    x = feat[...] * (att * imp[...])
    m = jnp.mean(x, axis=-1, keepdims=True)
    v = jnp.mean((x - m) * (x - m), axis=-1, keepdims=True)
    x = (x - m) / jnp.sqrt(v + 1e-5) * lng[...] + lnb[...]
    x = jax.nn.relu(jnp.dot(x, e1t[...], preferred_element_type=jnp.float32) + eb1[...])
    x = jax.nn.relu(jnp.dot(x, e2t[...], preferred_element_type=jnp.float32) + eb2[...])
    p_ref[...] = jnp.dot(x, wlt[...], preferred_element_type=jnp.float32)
    r_ref[...] = jnp.dot(x, wrt[...], preferred_element_type=jnp.float32) + sb[...]


def _mid1_body(sa, sb_, r, c0, c1, bng, bnb, wlt, wrt, b_, p_ref, r_ref, inv_ref):
    cnt = c0[...][:, 0:1] + c1[...][:, 0:1]
    inv = 1.0 / jnp.maximum(cnt, 1.0)
    s = jnp.concatenate([sa[...], sb_[...]], axis=1)
    x = jax.nn.relu((s * inv + r[...]) * (bng[...] * _BN_SCALE) + bnb[...])
    p_ref[...] = jnp.dot(x, wlt[...], preferred_element_type=jnp.float32)
    r_ref[...] = jnp.dot(x, wrt[...], preferred_element_type=jnp.float32) + b_[...]
    inv_ref[...] = inv


def _mid2_body(sa, sb_, r, inv, bng, bnb, wlt, wrt, b_, p_ref, r_ref):
    s = jnp.concatenate([sa[...], sb_[...]], axis=1)
    x = jax.nn.relu((s * inv[...] + r[...]) * (bng[...] * _BN_SCALE) + bnb[...])
    p_ref[...] = jnp.dot(x, wlt[...], preferred_element_type=jnp.float32)
    r_ref[...] = jnp.dot(x, wrt[...], preferred_element_type=jnp.float32) + b_[...]


def _final_body(sa, sb_, r, inv, bng, bnb, w1t, b1, w2t, b2, out_ref):
    s = jnp.concatenate([sa[...], sb_[...]], axis=1)
    x = jax.nn.relu((s * inv[...] + r[...]) * (bng[...] * _BN_SCALE) + bnb[...])
    h = jax.nn.relu(jnp.dot(x, w1t[...], preferred_element_type=jnp.float32) + b1[...])
    z = jnp.dot(h, w2t[...], preferred_element_type=jnp.float32) + b2[...]
    out_ref[...] = jax.nn.sigmoid(z)


def _make_agg(wh, ch):
    mesh = plsc.VectorSubcoreMesh(core_axis_name="c", subcore_axis_name="s")
    ept = _EPAD // _NT
    n_outer = ept // ch

    @functools.partial(
        pl.kernel,
        out_type=jax.ShapeDtypeStruct((2, _NS, wh), jnp.float32),
        mesh=mesh,
        scratch_types=[
            pltpu.VMEM((ch,), jnp.int32),
            pltpu.VMEM((ch,), jnp.int32),
            pltpu.VMEM((ch, wh), jnp.float32),
            pltpu.VMEM_SHARED((_NS, wh), jnp.float32),
            pltpu.SemaphoreType.DMA,
        ],
        compiler_params=pltpu.CompilerParams(use_tc_tiling_on_sc=False),
    )
    def agg(p_hbm, srcA, srcB, dst, zeros_hbm, out_hbm, src_v, dst_v, rows_v, acc, sem):
        c = lax.axis_index("c")
        s = lax.axis_index("s")
        pltpu.sync_copy(zeros_hbm, acc.at[pl.ds(s * _RPT, _RPT)])
        plsc.subcore_barrier()
        e0 = s * ept

        def outer(k, carry):
            b = e0 + k * ch

            @pl.when(c == 0)
            def _():
                pltpu.sync_copy(srcA.at[pl.ds(b, ch)], src_v)

            @pl.when(c == 1)
            def _():
                pltpu.sync_copy(srcB.at[pl.ds(b, ch)], src_v)

            pltpu.sync_copy(dst.at[pl.ds(b, ch)], dst_v)
            pltpu.async_copy(p_hbm.at[src_v], rows_v, sem).wait()
            pltpu.sync_copy(rows_v, acc.at[dst_v], add=True)
            return carry

        lax.fori_loop(0, n_outer, outer, 0)
        plsc.subcore_barrier()
        pltpu.sync_copy(acc.at[pl.ds(s * _RPT, _RPT)],
                        out_hbm.at[c, pl.ds(s * _RPT, _RPT)])

    return agg


def _make_cnt(ch):
    mesh = plsc.VectorSubcoreMesh(core_axis_name="c", subcore_axis_name="s")
    ept = _EPAD // 2 // _NT
    n_outer = ept // ch

    @functools.partial(
        pl.kernel,
        out_type=jax.ShapeDtypeStruct((2, _NS, 16), jnp.float32),
        mesh=mesh,
        scratch_types=[
            pltpu.VMEM((ch,), jnp.int32),
            pltpu.VMEM((ch, 16), jnp.float32),
            pltpu.VMEM_SHARED((_NS, 16), jnp.float32),
        ],
        compiler_params=pltpu.CompilerParams(use_tc_tiling_on_sc=False),
    )
    def cnt_k(dst, ones_hbm, zeros_hbm, out_hbm, dst_v, ones_v, acc):
        c = lax.axis_index("c")
        s = lax.axis_index("s")
        pltpu.sync_copy(ones_hbm, ones_v)
        pltpu.sync_copy(zeros_hbm, acc.at[pl.ds(s * _RPT, _RPT)])
        plsc.subcore_barrier()
        e0 = c * (_EPAD // 2) + s * ept

        def outer(k, carry):
            b = e0 + k * ch
            pltpu.sync_copy(dst.at[pl.ds(b, ch)], dst_v)
            pltpu.sync_copy(ones_v, acc.at[dst_v], add=True)
            return carry

        lax.fori_loop(0, n_outer, outer, 0)
        plsc.subcore_barrier()
        pltpu.sync_copy(acc.at[pl.ds(s * _RPT, _RPT)],
                        out_hbm.at[c, pl.ds(s * _RPT, _RPT)])

    return cnt_k


def kernel(accessibility_features, edge_index, context_features, ctx_W1, ctx_b1,
           ctx_W2, ctx_b2, att_W, att_b, base_importance, ln_g, ln_b,
           enc_W1, enc_b1, enc_W2, enc_b2, sage1_Wl, sage1_Wr, sage1_b,
           bn1_g, bn1_b, sage2_Wl, sage2_Wr, sage2_b, bn2_g, bn2_b,
           sage3_Wl, sage3_Wr, sage3_b, bn3_g, bn3_b,
           svi_W1, svi_b1, svi_W2, svi_b2):
    f32 = jnp.float32
    r1c = lambda a: a.reshape(1, -1).astype(f32)

    ctx = jnp.pad(context_features, ((0, 0), (0, 3)))
    w1t = jnp.pad(ctx_W1.T, ((0, 3), (0, 0)))
    src = edge_index[0]
    dst = edge_index[1]
    pad = _EPAD - _E
    srcA = jnp.concatenate([src * 2, jnp.zeros((pad,), jnp.int32)])
    srcB = jnp.concatenate([src * 2 + 1, jnp.zeros((pad,), jnp.int32)])
    dstP = jnp.concatenate([dst, jnp.full((pad,), _TRASH, jnp.int32)])
    zeros32 = jnp.zeros((_RPT, 32), f32)
    zeros16 = jnp.zeros((_RPT, 16), f32)
    ones16 = jnp.ones((3200, 16), f32)

    p1, r1 = pl.pallas_call(
        _pre_body,
        grid=(_GRID,),
        in_specs=[_rows(128), _rows(8), _full((8, 32)), _full((1, 32)),
                  _full((32, 32)), _full((1, 32)), _full((32, 128)), _full((1, 128)),
                  _full((1, 128)), _full((1, 128)), _full((1, 128)),
                  _full((128, 64)), _full((1, 64)), _full((64, 64)), _full((1, 64)),
                  _full((64, 64)), _full((64, 64)), _full((1, 64))],
        out_specs=[_rows(64), _rows(64)],
        out_shape=[jax.ShapeDtypeStruct((_N, 64), f32),
                   jax.ShapeDtypeStruct((_N, 64), f32)],
    )(accessibility_features, ctx, w1t, r1c(ctx_b1), ctx_W2.T, r1c(ctx_b2),
      att_W.T, r1c(att_b), r1c(base_importance), r1c(ln_g), r1c(ln_b),
      enc_W1.T, r1c(enc_b1), enc_W2.T, r1c(enc_b2),
      sage1_Wl.T, sage1_Wr.T, r1c(sage1_b))

    cnt2 = _make_cnt(3200)(dstP, ones16, zeros16)

    agg64 = _make_agg(32, 800)
    s1 = agg64(p1.reshape(2 * _N, 32), srcA, srcB, dstP, zeros32)

    p2, r2, inv = pl.pallas_call(
        _mid1_body,
        grid=(_GRID,),
        in_specs=[_rows(32), _rows(32), _rows(64), _rows(16), _rows(16),
                  _full((1, 64)), _full((1, 64)),
                  _full((64, 64)), _full((64, 64)), _full((1, 64))],
        out_specs=[_rows(64), _rows(64), _rows(1)],
        out_shape=[jax.ShapeDtypeStruct((_N, 64), f32),
                   jax.ShapeDtypeStruct((_N, 64), f32),
                   jax.ShapeDtypeStruct((_N, 1), f32)],
    )(s1[0], s1[1], r1, cnt2[0], cnt2[1], r1c(bn1_g), r1c(bn1_b),
      sage2_Wl.T, sage2_Wr.T, r1c(sage2_b))

    s2 = agg64(p2.reshape(2 * _N, 32), srcA, srcB, dstP, zeros32)

    p3, r3 = pl.pallas_call(
        _mid2_body,
        grid=(_GRID,),
        in_specs=[_rows(32), _rows(32), _rows(64), _rows(1),
                  _full((1, 64)), _full((1, 64)),
                  _full((64, 32)), _full((64, 32)), _full((1, 32))],
        out_specs=[_rows(32), _rows(32)],
        out_shape=[jax.ShapeDtypeStruct((_N, 32), f32),
                   jax.ShapeDtypeStruct((_N, 32), f32)],
    )(s2[0], s2[1], r2, inv, r1c(bn2_g), r1c(bn2_b),
      sage3_Wl.T, sage3_Wr.T, r1c(sage3_b))

    s3 = _make_agg(16, 3200)(p3.reshape(2 * _N, 16), srcA, srcB, dstP, zeros16)

    svi = pl.pallas_call(
        _final_body,
        grid=(_GRID,),
        in_specs=[_rows(16), _rows(16), _rows(32), _rows(1),
                  _full((1, 32)), _full((1, 32)),
                  _full((32, 16)), _full((1, 16)), _full((16, 1)), _full((1, 1))],
        out_specs=[_rows(1)],
        out_shape=[jax.ShapeDtypeStruct((_N, 1), f32)],
    )(s3[0], s3[1], r3, inv, r1c(bn3_g), r1c(bn3_b),
      svi_W1.T, r1c(svi_b1), svi_W2.T, r1c(svi_b2))[0]

    return svi[:, 0]


# speedup vs baseline: 5.1923x; 1.0380x over previous
import functools
import math

import jax
import jax.numpy as jnp
from jax import lax
from jax.experimental import pallas as pl
from jax.experimental.pallas import tpu as pltpu
from jax.experimental.pallas import tpu_sc as plsc

_BN_SCALE = 1.0 / math.sqrt(1.0 + 1e-5)

_N = 50000
_E = 800000
_BLK = 1000
_GRID = _N // _BLK
_NT = 16
_NS = 50048
_RPT = _NS // _NT
_EPAD = 819200
_TRASH = _N


def _full(spec_shape):
    nd = len(spec_shape)
    return pl.BlockSpec(spec_shape, lambda i, _n=nd: (0,) * _n)


def _rows(w):
    return pl.BlockSpec((_BLK, w), lambda i: (i, 0))


def _pre_body(feat, ctx, w1t, b1, w2t, b2, awt, ab, imp, lng, lnb,
              e1t, eb1, e2t, eb2, wlt, wrt, sb, p_ref, r_ref):
    ce = jax.nn.relu(jnp.dot(ctx[...], w1t[...], preferred_element_type=jnp.float32) + b1[...])
    ce = jnp.dot(ce, w2t[...], preferred_element_type=jnp.float32) + b2[...]
    logits = jnp.dot(ce, awt[...], preferred_element_type=jnp.float32) + ab[...]
    att = jax.nn.softmax(logits, axis=-1)
    x = feat[...] * (att * imp[...])
    m = jnp.mean(x, axis=-1, keepdims=True)
    v = jnp.mean((x - m) * (x - m), axis=-1, keepdims=True)
    x = (x - m) / jnp.sqrt(v + 1e-5) * lng[...] + lnb[...]
    x = jax.nn.relu(jnp.dot(x, e1t[...], preferred_element_type=jnp.float32) + eb1[...])
    x = jax.nn.relu(jnp.dot(x, e2t[...], preferred_element_type=jnp.float32) + eb2[...])
    p_ref[...] = jnp.dot(x, wlt[...], preferred_element_type=jnp.float32)
    r_ref[...] = jnp.dot(x, wrt[...], preferred_element_type=jnp.float32) + sb[...]


def _mid1_body(sa, sb_, r, c0, c1, bng, bnb, wlt, wrt, b_, p_ref, r_ref, inv_ref):
    cnt = c0[...][:, 0:1] + c1[...][:, 0:1]
    inv = 1.0 / jnp.maximum(cnt, 1.0)
    s = jnp.concatenate([sa[...], sb_[...]], axis=1)
    x = jax.nn.relu((s * inv + r[...]) * (bng[...] * _BN_SCALE) + bnb[...])
    p_ref[...] = jnp.dot(x, wlt[...], preferred_element_type=jnp.float32)
    r_ref[...] = jnp.dot(x, wrt[...], preferred_element_type=jnp.float32) + b_[...]
    inv_ref[...] = inv


def _mid2_body(sa, sb_, r, inv, bng, bnb, wlt, wrt, b_, p_ref, r_ref):
    s = jnp.concatenate([sa[...], sb_[...]], axis=1)
    x = jax.nn.relu((s * inv[...] + r[...]) * (bng[...] * _BN_SCALE) + bnb[...])
    p_ref[...] = jnp.dot(x, wlt[...], preferred_element_type=jnp.float32)
    r_ref[...] = jnp.dot(x, wrt[...], preferred_element_type=jnp.float32) + b_[...]


def _final_body(sa, sb_, r, inv, bng, bnb, w1t, b1, w2t, b2, out_ref):
    s = jnp.concatenate([sa[...], sb_[...]], axis=1)
    x = jax.nn.relu((s * inv[...] + r[...]) * (bng[...] * _BN_SCALE) + bnb[...])
    h = jax.nn.relu(jnp.dot(x, w1t[...], preferred_element_type=jnp.float32) + b1[...])
    z = jnp.dot(h, w2t[...], preferred_element_type=jnp.float32) + b2[...]
    out_ref[...] = jax.nn.sigmoid(z)


def _make_agg(wh, ch, m_per_slab):
    mesh = plsc.VectorSubcoreMesh(core_axis_name="c", subcore_axis_name="s")
    ept = _EPAD // _NT
    n_chunks = ept // ch
    n_slab = n_chunks // m_per_slab
    M = m_per_slab

    @functools.partial(
        pl.kernel,
        out_type=jax.ShapeDtypeStruct((2, _NS, wh), jnp.float32),
        mesh=mesh,
        scratch_types=[
            pltpu.VMEM((M, ch), jnp.int32),
            pltpu.VMEM((M, ch), jnp.int32),
            pltpu.VMEM((ch, wh), jnp.float32),
            pltpu.VMEM((ch, wh), jnp.float32),
            pltpu.VMEM_SHARED((_NS, wh), jnp.float32),
            pltpu.SemaphoreType.DMA,
            pltpu.SemaphoreType.DMA,
            pltpu.SemaphoreType.DMA,
            pltpu.SemaphoreType.DMA,
        ],
        compiler_params=pltpu.CompilerParams(use_tc_tiling_on_sc=False),
    )
    def agg(p_hbm, srcA, srcB, dst, zeros_hbm, out_hbm,
            src2, dst2, rows0, rows1, acc, g0, g1, s0, s1):
        c = lax.axis_index("c")
        s = lax.axis_index("s")
        pltpu.sync_copy(zeros_hbm, acc.at[pl.ds(s * _RPT, _RPT)])
        plsc.subcore_barrier()
        r0 = s * n_chunks
        rbuf = (rows0, rows1)
        gsem = (g0, g1)
        ssem = (s0, s1)

        def slab(t, carry):
            row = r0 + t * M

            @pl.when(c == 0)
            def _():
                pltpu.sync_copy(srcA.at[pl.ds(row, M)], src2)

            @pl.when(c == 1)
            def _():
                pltpu.sync_copy(srcB.at[pl.ds(row, M)], src2)

            pltpu.sync_copy(dst.at[pl.ds(row, M)], dst2)
            h_g = [None] * M
            h_s = [None] * M
            h_g[0] = pltpu.async_copy(p_hbm.at[src2.at[0]], rbuf[0], gsem[0])
            for m in range(M):
                h_g[m].wait()
                if m + 1 < M:
                    if m >= 1:
                        h_s[m - 1].wait()
                    h_g[m + 1] = pltpu.async_copy(
                        p_hbm.at[src2.at[m + 1]], rbuf[(m + 1) % 2], gsem[(m + 1) % 2])
                h_s[m] = pltpu.async_copy(
                    rbuf[m % 2], acc.at[dst2.at[m]], ssem[m % 2], add=True)
            if M >= 2:
                h_s[M - 2].wait()
            h_s[M - 1].wait()
            return carry

        lax.fori_loop(0, n_slab, slab, 0)
        plsc.subcore_barrier()
        pltpu.sync_copy(acc.at[pl.ds(s * _RPT, _RPT)],
                        out_hbm.at[c, pl.ds(s * _RPT, _RPT)])

    return agg


def _make_cnt(ch):
    mesh = plsc.VectorSubcoreMesh(core_axis_name="c", subcore_axis_name="s")
    ept = _EPAD // 2 // _NT
    n_outer = ept // ch

    @functools.partial(
        pl.kernel,
        out_type=jax.ShapeDtypeStruct((2, _NS, 16), jnp.float32),
        mesh=mesh,
        scratch_types=[
            pltpu.VMEM((ch,), jnp.int32),
            pltpu.VMEM((ch, 16), jnp.float32),
            pltpu.VMEM_SHARED((_NS, 16), jnp.float32),
        ],
        compiler_params=pltpu.CompilerParams(use_tc_tiling_on_sc=False),
    )
    def cnt_k(dst, ones_hbm, zeros_hbm, out_hbm, dst_v, ones_v, acc):
        c = lax.axis_index("c")
        s = lax.axis_index("s")
        pltpu.sync_copy(ones_hbm, ones_v)
        pltpu.sync_copy(zeros_hbm, acc.at[pl.ds(s * _RPT, _RPT)])
        plsc.subcore_barrier()
        e0 = c * (_EPAD // 2) + s * ept

        def outer(k, carry):
            b = e0 + k * ch
            pltpu.sync_copy(dst.at[pl.ds(b, ch)], dst_v)
            pltpu.sync_copy(ones_v, acc.at[dst_v], add=True)
            return carry

        lax.fori_loop(0, n_outer, outer, 0)
        plsc.subcore_barrier()
        pltpu.sync_copy(acc.at[pl.ds(s * _RPT, _RPT)],
                        out_hbm.at[c, pl.ds(s * _RPT, _RPT)])

    return cnt_k


def kernel(accessibility_features, edge_index, context_features, ctx_W1, ctx_b1,
           ctx_W2, ctx_b2, att_W, att_b, base_importance, ln_g, ln_b,
           enc_W1, enc_b1, enc_W2, enc_b2, sage1_Wl, sage1_Wr, sage1_b,
           bn1_g, bn1_b, sage2_Wl, sage2_Wr, sage2_b, bn2_g, bn2_b,
           sage3_Wl, sage3_Wr, sage3_b, bn3_g, bn3_b,
           svi_W1, svi_b1, svi_W2, svi_b2):
    f32 = jnp.float32
    r1c = lambda a: a.reshape(1, -1).astype(f32)

    ctx = jnp.pad(context_features, ((0, 0), (0, 3)))
    w1t = jnp.pad(ctx_W1.T, ((0, 3), (0, 0)))
    src = edge_index[0]
    dst = edge_index[1]
    pad = _EPAD - _E
    srcA = jnp.concatenate([src * 2, jnp.zeros((pad,), jnp.int32)])
    srcB = jnp.concatenate([src * 2 + 1, jnp.zeros((pad,), jnp.int32)])
    dstP = jnp.concatenate([dst, jnp.full((pad,), _TRASH, jnp.int32)])
    zeros32 = jnp.zeros((_RPT, 32), f32)
    zeros16 = jnp.zeros((_RPT, 16), f32)
    ones16 = jnp.ones((3200, 16), f32)

    p1, r1 = pl.pallas_call(
        _pre_body,
        grid=(_GRID,),
        in_specs=[_rows(128), _rows(8), _full((8, 32)), _full((1, 32)),
                  _full((32, 32)), _full((1, 32)), _full((32, 128)), _full((1, 128)),
                  _full((1, 128)), _full((1, 128)), _full((1, 128)),
                  _full((128, 64)), _full((1, 64)), _full((64, 64)), _full((1, 64)),
                  _full((64, 64)), _full((64, 64)), _full((1, 64))],
        out_specs=[_rows(64), _rows(64)],
        out_shape=[jax.ShapeDtypeStruct((_N, 64), f32),
                   jax.ShapeDtypeStruct((_N, 64), f32)],
    )(accessibility_features, ctx, w1t, r1c(ctx_b1), ctx_W2.T, r1c(ctx_b2),
      att_W.T, r1c(att_b), r1c(base_importance), r1c(ln_g), r1c(ln_b),
      enc_W1.T, r1c(enc_b1), enc_W2.T, r1c(enc_b2),
      sage1_Wl.T, sage1_Wr.T, r1c(sage1_b))

    cnt2 = _make_cnt(3200)(dstP, ones16, zeros16)

    sA320 = srcA.reshape(_EPAD // 320, 320)
    sB320 = srcB.reshape(_EPAD // 320, 320)
    dP320 = dstP.reshape(_EPAD // 320, 320)
    sA1600 = srcA.reshape(_EPAD // 1600, 1600)
    sB1600 = srcB.reshape(_EPAD // 1600, 1600)
    dP1600 = dstP.reshape(_EPAD // 1600, 1600)
    agg64 = _make_agg(32, 320, 10)
    s1 = agg64(p1.reshape(2 * _N, 32), sA320, sB320, dP320, zeros32)

    p2, r2, inv = pl.pallas_call(
        _mid1_body,
        grid=(_GRID,),
        in_specs=[_rows(32), _rows(32), _rows(64), _rows(16), _rows(16),
                  _full((1, 64)), _full((1, 64)),
                  _full((64, 64)), _full((64, 64)), _full((1, 64))],
        out_specs=[_rows(64), _rows(64), _rows(1)],
        out_shape=[jax.ShapeDtypeStruct((_N, 64), f32),
                   jax.ShapeDtypeStruct((_N, 64), f32),
                   jax.ShapeDtypeStruct((_N, 1), f32)],
    )(s1[0], s1[1], r1, cnt2[0], cnt2[1], r1c(bn1_g), r1c(bn1_b),
      sage2_Wl.T, sage2_Wr.T, r1c(sage2_b))

    s2 = agg64(p2.reshape(2 * _N, 32), sA320, sB320, dP320, zeros32)

    p3, r3 = pl.pallas_call(
        _mid2_body,
        grid=(_GRID,),
        in_specs=[_rows(32), _rows(32), _rows(64), _rows(1),
                  _full((1, 64)), _full((1, 64)),
                  _full((64, 32)), _full((64, 32)), _full((1, 32))],
        out_specs=[_rows(32), _rows(32)],
        out_shape=[jax.ShapeDtypeStruct((_N, 32), f32),
                   jax.ShapeDtypeStruct((_N, 32), f32)],
    )(s2[0], s2[1], r2, inv, r1c(bn2_g), r1c(bn2_b),
      sage3_Wl.T, sage3_Wr.T, r1c(sage3_b))

    s3 = _make_agg(16, 1600, 8)(p3.reshape(2 * _N, 16), sA1600, sB1600, dP1600, zeros16)

    svi = pl.pallas_call(
        _final_body,
        grid=(_GRID,),
        in_specs=[_rows(16), _rows(16), _rows(32), _rows(1),
                  _full((1, 32)), _full((1, 32)),
                  _full((32, 16)), _full((1, 16)), _full((16, 1)), _full((1, 1))],
        out_specs=[_rows(1)],
        out_shape=[jax.ShapeDtypeStruct((_N, 1), f32)],
    )(s3[0], s3[1], r3, inv, r1c(bn3_g), r1c(bn3_b),
      svi_W1.T, r1c(svi_b1), svi_W2.T, r1c(svi_b2))[0]

    return svi[:, 0]


# speedup vs baseline: 9.3770x; 1.8059x over previous
import functools
import math

import jax
import jax.numpy as jnp
from jax import lax
from jax.experimental import pallas as pl
from jax.experimental.pallas import tpu as pltpu
from jax.experimental.pallas import tpu_sc as plsc

_BN_SCALE = 1.0 / math.sqrt(1.0 + 1e-5)

_N = 50000
_E = 800000
_BLK = 1000
_GRID = _N // _BLK
_NT = 16
_NS = 50048
_RPT = _NS // _NT
_EPAD = 819200
_TRASH = _N


def _full(spec_shape):
    nd = len(spec_shape)
    return pl.BlockSpec(spec_shape, lambda i, _n=nd: (0,) * _n)


def _rows(w):
    return pl.BlockSpec((_BLK, w), lambda i: (i, 0))


def _pre_body(feat, ctx, w1t, b1, w2t, b2, awt, ab, imp, lng, lnb,
              e1t, eb1, e2t, eb2, wlt, wrt, sb, p_ref, r_ref):
    ce = jax.nn.relu(jnp.dot(ctx[...], w1t[...], preferred_element_type=jnp.float32) + b1[...])
    ce = jnp.dot(ce, w2t[...], preferred_element_type=jnp.float32) + b2[...]
    logits = jnp.dot(ce, awt[...], preferred_element_type=jnp.float32) + ab[...]
    att = jax.nn.softmax(logits, axis=-1)
    x = feat[...] * (att * imp[...])
    m = jnp.mean(x, axis=-1, keepdims=True)
    v = jnp.mean((x - m) * (x - m), axis=-1, keepdims=True)
    x = (x - m) / jnp.sqrt(v + 1e-5) * lng[...] + lnb[...]
    x = jax.nn.relu(jnp.dot(x, e1t[...], preferred_element_type=jnp.float32) + eb1[...])
    x = jax.nn.relu(jnp.dot(x, e2t[...], preferred_element_type=jnp.float32) + eb2[...])
    p_ref[...] = jnp.dot(x, wlt[...], preferred_element_type=jnp.float32)
    r_ref[...] = jnp.dot(x, wrt[...], preferred_element_type=jnp.float32) + sb[...]


def _mid1_body(sa, sb_, r, c0, c1, bng, bnb, wlt, wrt, b_, p_ref, r_ref, inv_ref):
    cnt = c0[...][:, 0:1] + c1[...][:, 0:1]
    inv = 1.0 / jnp.maximum(cnt, 1.0)
    s = jnp.concatenate([sa[...], sb_[...]], axis=1)
    x = jax.nn.relu((s * inv + r[...]) * (bng[...] * _BN_SCALE) + bnb[...])
    p_ref[...] = jnp.dot(x, wlt[...], preferred_element_type=jnp.float32)
    r_ref[...] = jnp.dot(x, wrt[...], preferred_element_type=jnp.float32) + b_[...]
    inv_ref[...] = inv


def _mid2_body(sa, sb_, r, inv, bng, bnb, wlt, wrt, b_, p_ref, r_ref):
    s = jnp.concatenate([sa[...], sb_[...]], axis=1)
    x = jax.nn.relu((s * inv[...] + r[...]) * (bng[...] * _BN_SCALE) + bnb[...])
    p_ref[...] = jnp.dot(x, wlt[...], preferred_element_type=jnp.float32)
    r_ref[...] = jnp.dot(x, wrt[...], preferred_element_type=jnp.float32) + b_[...]


def _final_body(sa, sb_, r, inv, bng, bnb, w1t, b1, w2t, b2, out_ref):
    s = jnp.concatenate([sa[...], sb_[...]], axis=1)
    x = jax.nn.relu((s * inv[...] + r[...]) * (bng[...] * _BN_SCALE) + bnb[...])
    h = jax.nn.relu(jnp.dot(x, w1t[...], preferred_element_type=jnp.float32) + b1[...])
    z = jnp.dot(h, w2t[...], preferred_element_type=jnp.float32) + b2[...]
    out_ref[...] = jax.nn.sigmoid(z)


def _make_agg(wh, ch, m_per_slab):
    mesh = plsc.VectorSubcoreMesh(core_axis_name="c", subcore_axis_name="s")
    ept = _E // _NT
    n_chunks = ept // ch
    n_slab = n_chunks // m_per_slab
    M = m_per_slab

    @functools.partial(
        pl.kernel,
        out_type=[jax.ShapeDtypeStruct((_NS, wh), jnp.float32),
                  jax.ShapeDtypeStruct((_NS, wh), jnp.float32)],
        mesh=mesh,
        scratch_types=[
            pltpu.VMEM((M, ch), jnp.int32),
            pltpu.VMEM((M, ch), jnp.int32),
            pltpu.VMEM((ch, wh), jnp.float32),
            pltpu.VMEM((ch, wh), jnp.float32),
            pltpu.VMEM_SHARED((_NS, wh), jnp.float32),
            pltpu.SemaphoreType.DMA,
            pltpu.SemaphoreType.DMA,
            pltpu.SemaphoreType.DMA,
            pltpu.SemaphoreType.DMA,
        ],
        compiler_params=pltpu.CompilerParams(use_tc_tiling_on_sc=False),
    )
    def agg(p_hbm, srcA, srcB, dst, zeros_hbm, out0, out1,
            src2, dst2, rows0, rows1, acc, g0, g1, s0, s1):
        c = lax.axis_index("c")
        s = lax.axis_index("s")
        pltpu.sync_copy(zeros_hbm, acc.at[pl.ds(s * _RPT, _RPT)])
        plsc.subcore_barrier()
        r0 = s * n_chunks
        rbuf = (rows0, rows1)
        gsem = (g0, g1)
        ssem = (s0, s1)

        def slab(t, carry):
            row = r0 + t * M

            @pl.when(c == 0)
            def _():
                pltpu.sync_copy(srcA.at[pl.ds(row, M)], src2)

            @pl.when(c == 1)
            def _():
                pltpu.sync_copy(srcB.at[pl.ds(row, M)], src2)

            pltpu.sync_copy(dst.at[pl.ds(row, M)], dst2)
            h_g = [None] * M
            h_s = [None] * M
            h_g[0] = pltpu.async_copy(p_hbm.at[src2.at[0]], rbuf[0], gsem[0])
            for m in range(M):
                h_g[m].wait()
                if m + 1 < M:
                    if m >= 1:
                        h_s[m - 1].wait()
                    h_g[m + 1] = pltpu.async_copy(
                        p_hbm.at[src2.at[m + 1]], rbuf[(m + 1) % 2], gsem[(m + 1) % 2])
                h_s[m] = pltpu.async_copy(
                    rbuf[m % 2], acc.at[dst2.at[m]], ssem[m % 2], add=True)
            if M >= 2:
                h_s[M - 2].wait()
            h_s[M - 1].wait()
            return carry

        lax.fori_loop(0, n_slab, slab, 0)
        plsc.subcore_barrier()

        @pl.when(c == 0)
        def _():
            pltpu.sync_copy(acc.at[pl.ds(s * _RPT, _RPT)],
                            out0.at[pl.ds(s * _RPT, _RPT)])

        @pl.when(c == 1)
        def _():
            pltpu.sync_copy(acc.at[pl.ds(s * _RPT, _RPT)],
                            out1.at[pl.ds(s * _RPT, _RPT)])

    return agg


def _make_cnt(ch):
    mesh = plsc.VectorSubcoreMesh(core_axis_name="c", subcore_axis_name="s")
    ept = _E // 2 // _NT
    n_chunks = ept // ch

    @functools.partial(
        pl.kernel,
        out_type=[jax.ShapeDtypeStruct((_NS, 16), jnp.float32),
                  jax.ShapeDtypeStruct((_NS, 16), jnp.float32)],
        mesh=mesh,
        scratch_types=[
            pltpu.VMEM((n_chunks, ch), jnp.int32),
            pltpu.VMEM((ch, 16), jnp.float32),
            pltpu.VMEM_SHARED((_NS, 16), jnp.float32),
            pltpu.SemaphoreType.DMA,
        ],
        compiler_params=pltpu.CompilerParams(use_tc_tiling_on_sc=False),
    )
    def cnt_k(dst, ones_hbm, zeros_hbm, out0, out1, dst2, ones_v, acc, sem):
        c = lax.axis_index("c")
        s = lax.axis_index("s")
        pltpu.sync_copy(ones_hbm, ones_v)
        pltpu.sync_copy(zeros_hbm, acc.at[pl.ds(s * _RPT, _RPT)])
        plsc.subcore_barrier()
        r0 = (c * (_E // 2) + s * ept) // ch
        pltpu.sync_copy(dst.at[pl.ds(r0, n_chunks)], dst2)
        hs = [pltpu.async_copy(ones_v, acc.at[dst2.at[m]], sem, add=True)
              for m in range(n_chunks)]
        for h in hs:
            h.wait()
        plsc.subcore_barrier()

        @pl.when(c == 0)
        def _():
            pltpu.sync_copy(acc.at[pl.ds(s * _RPT, _RPT)],
                            out0.at[pl.ds(s * _RPT, _RPT)])

        @pl.when(c == 1)
        def _():
            pltpu.sync_copy(acc.at[pl.ds(s * _RPT, _RPT)],
                            out1.at[pl.ds(s * _RPT, _RPT)])

    return cnt_k


def kernel(accessibility_features, edge_index, context_features, ctx_W1, ctx_b1,
           ctx_W2, ctx_b2, att_W, att_b, base_importance, ln_g, ln_b,
           enc_W1, enc_b1, enc_W2, enc_b2, sage1_Wl, sage1_Wr, sage1_b,
           bn1_g, bn1_b, sage2_Wl, sage2_Wr, sage2_b, bn2_g, bn2_b,
           sage3_Wl, sage3_Wr, sage3_b, bn3_g, bn3_b,
           svi_W1, svi_b1, svi_W2, svi_b2):
    f32 = jnp.float32
    r1c = lambda a: a.reshape(1, -1).astype(f32)

    ctx = jnp.pad(context_features, ((0, 0), (0, 3)))
    w1t = jnp.pad(ctx_W1.T, ((0, 3), (0, 0)))
    src = edge_index[0]
    dst = edge_index[1]
    srcA = src * 2
    srcB = srcA + 1
    zeros32 = jnp.zeros((_RPT, 32), f32)
    zeros16 = jnp.zeros((_RPT, 16), f32)
    ones16 = jnp.ones((1000, 16), f32)

    p1, r1 = pl.pallas_call(
        _pre_body,
        grid=(_GRID,),
        in_specs=[_rows(128), _rows(8), _full((8, 32)), _full((1, 32)),
                  _full((32, 32)), _full((1, 32)), _full((32, 128)), _full((1, 128)),
                  _full((1, 128)), _full((1, 128)), _full((1, 128)),
                  _full((128, 64)), _full((1, 64)), _full((64, 64)), _full((1, 64)),
                  _full((64, 64)), _full((64, 64)), _full((1, 64))],
        out_specs=[_rows(64), _rows(64)],
        out_shape=[jax.ShapeDtypeStruct((_N, 64), f32),
                   jax.ShapeDtypeStruct((_N, 64), f32)],
    )(accessibility_features, ctx, w1t, r1c(ctx_b1), ctx_W2.T, r1c(ctx_b2),
      att_W.T, r1c(att_b), r1c(base_importance), r1c(ln_g), r1c(ln_b),
      enc_W1.T, r1c(enc_b1), enc_W2.T, r1c(enc_b2),
      sage1_Wl.T, sage1_Wr.T, r1c(sage1_b))

    d1000 = dst.reshape(_E // 1000, 1000)
    cnt2 = _make_cnt(1000)(d1000, ones16, zeros16)

    sA400 = srcA.reshape(_E // 400, 400)
    sB400 = srcB.reshape(_E // 400, 400)
    d400 = dst.reshape(_E // 400, 400)
    sA1000 = srcA.reshape(_E // 1000, 1000)
    sB1000 = srcB.reshape(_E // 1000, 1000)
    agg64 = _make_agg(32, 400, 5)
    s1 = agg64(p1.reshape(2 * _N, 32), sA400, sB400, d400, zeros32)

    p2, r2, inv = pl.pallas_call(
        _mid1_body,
        grid=(_GRID,),
        in_specs=[_rows(32), _rows(32), _rows(64), _rows(16), _rows(16),
                  _full((1, 64)), _full((1, 64)),
                  _full((64, 64)), _full((64, 64)), _full((1, 64))],
        out_specs=[_rows(64), _rows(64), _rows(1)],
        out_shape=[jax.ShapeDtypeStruct((_N, 64), f32),
                   jax.ShapeDtypeStruct((_N, 64), f32),
                   jax.ShapeDtypeStruct((_N, 1), f32)],
    )(s1[0], s1[1], r1, cnt2[0], cnt2[1], r1c(bn1_g), r1c(bn1_b),
      sage2_Wl.T, sage2_Wr.T, r1c(sage2_b))

    s2 = agg64(p2.reshape(2 * _N, 32), sA400, sB400, d400, zeros32)

    p3, r3 = pl.pallas_call(
        _mid2_body,
        grid=(_GRID,),
        in_specs=[_rows(32), _rows(32), _rows(64), _rows(1),
                  _full((1, 64)), _full((1, 64)),
                  _full((64, 32)), _full((64, 32)), _full((1, 32))],
        out_specs=[_rows(32), _rows(32)],
        out_shape=[jax.ShapeDtypeStruct((_N, 32), f32),
                   jax.ShapeDtypeStruct((_N, 32), f32)],
    )(s2[0], s2[1], r2, inv, r1c(bn2_g), r1c(bn2_b),
      sage3_Wl.T, sage3_Wr.T, r1c(sage3_b))

    s3 = _make_agg(16, 1000, 10)(p3.reshape(2 * _N, 16), sA1000, sB1000, d1000, zeros16)

    svi = pl.pallas_call(
        _final_body,
        grid=(_GRID,),
        in_specs=[_rows(16), _rows(16), _rows(32), _rows(1),
                  _full((1, 32)), _full((1, 32)),
                  _full((32, 16)), _full((1, 16)), _full((16, 1)), _full((1, 1))],
        out_specs=[_rows(1)],
        out_shape=[jax.ShapeDtypeStruct((_N, 1), f32)],
    )(s3[0], s3[1], r3, inv, r1c(bn3_g), r1c(bn3_b),
      svi_W1.T, r1c(svi_b1), svi_W2.T, r1c(svi_b2))[0]

    return svi[:, 0]
